# baseline (speedup 1.0000x reference)
import functools

import jax
import jax.numpy as jnp
from jax import lax
from jax.experimental import pallas as pl
from jax.experimental.pallas import tpu as pltpu
from jax.experimental.pallas import tpu_sc as plsc

N = 10000
E = 160000
D = 256
ED = 16
H = 256
G = 64

NCORE = 2
NSUB = 16
HW = 128
CHUNK = 32
NC = 320
IGRP = 16
NGR = NC // IGRP
EPAD = NSUB * NC * CHUNK
NCH = EPAD // CHUNK
NSP = 10112
ZROWS = NSP // NSUB
OROWS = 632
OROWS_LAST = N - (NSUB - 1) * OROWS
EB = 2048
NB = 1000


def _edge_lin_body(ea, we1, be1, we2, be2, o1, o2):
    blk = ea[...]
    e1 = jnp.dot(blk, we1[0], preferred_element_type=jnp.float32) + be1[0]
    e2 = jnp.dot(blk, we2[0], preferred_element_type=jnp.float32) + be2[0]
    o1[...] = e1.reshape(EB // CHUNK, CHUNK, HW)
    o2[...] = e2.reshape(EB // CHUNK, CHUNK, HW)


def _edge_lin(ea_p, W_e1, b_e1, W_e2, b_e2):
    nprog = EPAD // EB
    grid = (nprog, NCORE)
    out_shape = jax.ShapeDtypeStruct((NCORE * NCH, CHUNK, HW), jnp.float32)
    return pl.pallas_call(
        _edge_lin_body,
        grid=grid,
        in_specs=[
            pl.BlockSpec((EB, ED), lambda i, c: (i, 0)),
            pl.BlockSpec((1, ED, HW), lambda i, c: (c, 0, 0)),
            pl.BlockSpec((1, 1, HW), lambda i, c: (c, 0, 0)),
            pl.BlockSpec((1, ED, HW), lambda i, c: (c, 0, 0)),
            pl.BlockSpec((1, 1, HW), lambda i, c: (c, 0, 0)),
        ],
        out_specs=[
            pl.BlockSpec((EB // CHUNK, CHUNK, HW), lambda i, c: (c * nprog + i, 0, 0)),
            pl.BlockSpec((EB // CHUNK, CHUNK, HW), lambda i, c: (c * nprog + i, 0, 0)),
        ],
        out_shape=[out_shape, out_shape],
    )(ea_p, W_e1.reshape(ED, NCORE, HW).transpose(1, 0, 2),
      b_e1.reshape(NCORE, 1, HW),
      W_e2.reshape(ED, NCORE, HW).transpose(1, 0, 2),
      b_e2.reshape(NCORE, 1, HW))


def _sc_msg_body(xi, e4, src3, dst3, out_lo, out_hi,
                 idx_a, idx_b, dst_v,
                 eb0, eb1, eb2, eb3, xb0, xb1, xb2, xb3, mb0, mb1,
                 agg_sh,
                 se0, se1, se2, se3, ss0, ss1):
    c = lax.axis_index("c")
    sid = lax.axis_index("s")
    ebs, xbs = (eb0, eb1, eb2, eb3), (xb0, xb1, xb2, xb3)
    ses = (se0, se1, se2, se3)
    mbs, sss = (mb0, mb1), (ss0, ss1)
    ebase = c * NCH + sid * NC

    @plsc.parallel_loop(0, CHUNK)
    def _(r):
        for k in range(HW // 16):
            mb0[r, pl.ds(k * 16, 16)] = jnp.zeros((16,), jnp.float32)
    for z in range(ZROWS // CHUNK):
        pltpu.sync_copy(mb0, agg_sh.at[pl.ds(sid * ZROWS + z * CHUNK, CHUNK)])
    zrem = ZROWS % CHUNK
    if zrem:
        pltpu.sync_copy(
            mb0.at[pl.ds(0, zrem)],
            agg_sh.at[pl.ds(sid * ZROWS + (ZROWS // CHUNK) * CHUNK, zrem)])

    def load_idx(buf, grp):
        pltpu.sync_copy(src3.at[sid * NGR + grp], buf)

        @plsc.parallel_loop(0, IGRP)
        def _(i):
            for k in range(CHUNK // 16):
                sl = pl.ds(k * 16, 16)
                buf[i, sl] = buf[i, sl] * 2 + c

    def wait_e(par):
        pltpu.make_async_copy(e4.at[ebase], ebs[par], ses[par]).wait()

    def wait_x(par):
        pltpu.make_async_copy(e4.at[ebase], xbs[par], ses[par]).wait()

    def wait_s(par):
        pltpu.make_async_copy(e4.at[ebase], mbs[par], sss[par]).wait()

    def step(grp, lcl, par4, par2, wait_sc, pf_row, pf_guard):
        wait_e(par4)
        wait_x(par4)
        if wait_sc:
            wait_s(par2)

        eb, xb, mb = ebs[par4], xbs[par4], mbs[par2]

        @plsc.parallel_loop(0, CHUNK, unroll=2)
        def _(r):
            for k in range(HW // 16):
                sl = pl.ds(k * 16, 16)
                mb[r, sl] = jnp.maximum(eb[r, sl] + xb[r, sl], 0.0)

        pltpu.async_copy(mb, agg_sh.at[dst_v.at[lcl]], sss[par2], add=True)

        if pf_row is not None:
            j2 = grp * IGRP + lcl + 4

            @pl.when(pf_guard)
            def _():
                pltpu.async_copy(e4.at[ebase + j2], ebs[par4], ses[par4])
                pltpu.async_copy(xi.at[pf_row], xbs[par4], ses[par4])

    load_idx(idx_a, 0)
    pltpu.sync_copy(dst3.at[sid * NGR], dst_v)

    plsc.subcore_barrier()

    for q in range(4):
        pltpu.async_copy(e4.at[ebase + q], ebs[q], ses[q])
        pltpu.async_copy(xi.at[idx_a.at[q]], xbs[q], ses[q])

    def quad(grp, base, gcur, wait_sc_head=True, pf_bufs=None,
             pf_guard=None):
        for q in range(4):
            lcl = base + q
            wsc = wait_sc_head if (q < 2) else True
            if pf_bufs is None:
                step(grp, lcl, q, q % 2, wsc, gcur.at[base + 4 + q],
                     jnp.bool_(True))
            else:
                step(grp, lcl, q, q % 2, wsc, pf_bufs[q], pf_guard)

    def run_group(grp, gcur, gnxt, last):
        quad(grp, 0, gcur, wait_sc_head=False)

        def _quad(t, _):
            quad(grp, 4 * t, gcur)
            return 0
        lax.fori_loop(1, IGRP // 4 - 1, _quad, 0)

        quad(grp, IGRP - 4, gcur,
             pf_bufs=[gnxt.at[q] for q in range(4)],
             pf_guard=jnp.logical_not(last))
        wait_s(0)
        wait_s(1)

        @pl.when(jnp.logical_not(last))
        def _():
            pltpu.sync_copy(dst3.at[sid * NGR + grp + 1], dst_v)

    def _gg(gg, _):
        g0 = 2 * gg
        load_idx(idx_b, g0 + 1)
        run_group(g0, idx_a, idx_b, jnp.bool_(False))

        @pl.when(gg < NGR // 2 - 1)
        def _():
            load_idx(idx_a, g0 + 2)
        run_group(g0 + 1, idx_b, idx_a, gg >= NGR // 2 - 1)
        return 0
    lax.fori_loop(0, NGR // 2, _gg, 0)

    plsc.subcore_barrier()

    base = sid * OROWS
    for cc, out in ((0, out_lo), (1, out_hi)):
        @pl.when(jnp.logical_and(c == cc, sid < NSUB - 1))
        def _(out=out):
            pltpu.sync_copy(agg_sh.at[pl.ds(base, OROWS)],
                            out.at[pl.ds(base, OROWS)])

        @pl.when(jnp.logical_and(c == cc, sid == NSUB - 1))
        def _(out=out):
            pltpu.sync_copy(agg_sh.at[pl.ds(base, OROWS_LAST)],
                            out.at[pl.ds(base, OROWS_LAST)])


_sc_msg = functools.partial(
    pl.kernel,
    out_type=(jax.ShapeDtypeStruct((N, HW), jnp.float32),
              jax.ShapeDtypeStruct((N, HW), jnp.float32)),
    mesh=plsc.VectorSubcoreMesh(core_axis_name="c", subcore_axis_name="s"),
    scratch_types=(
        [pltpu.VMEM((IGRP, CHUNK), jnp.int32) for _ in range(3)]
        + [pltpu.VMEM((CHUNK, HW), jnp.float32) for _ in range(10)]
        + [pltpu.VMEM_SHARED((NSP, HW), jnp.float32)]
        + [pltpu.SemaphoreType.DMA for _ in range(6)]
    ),
)(_sc_msg_body)


def _node_lin_body(xr, alo, ahi, wn, bn, out):
    xb = xr[...]
    xa = jnp.concatenate([xb[:, :HW] + alo[...], xb[:, HW:] + ahi[...]], axis=1)
    h = jnp.dot(xa, wn[...], preferred_element_type=jnp.float32) + bn[0]
    out[...] = h


def _node_lin(x, agg_lo, agg_hi, W_n, b_n):
    grid = (N // NB, NCORE)
    return pl.pallas_call(
        _node_lin_body,
        grid=grid,
        in_specs=[
            pl.BlockSpec((NB, D), lambda i, c: (i, 0)),
            pl.BlockSpec((NB, HW), lambda i, c: (i, 0)),
            pl.BlockSpec((NB, HW), lambda i, c: (i, 0)),
            pl.BlockSpec((D, HW), lambda i, c: (0, c)),
            pl.BlockSpec((1, 1, HW), lambda i, c: (c, 0, 0)),
        ],
        out_specs=pl.BlockSpec((NB, HW), lambda i, c: (i, c)),
        out_shape=jax.ShapeDtypeStruct((N, D), jnp.float32),
    )(x, agg_lo, agg_hi, W_n, b_n.reshape(NCORE, 1, HW))


def _final_body(h, alo, ahi, wn, bn, bt, wr1, br1, wr2, br2, out, acc):
    i = pl.program_id(0)
    hb = h[...]
    xa = jnp.concatenate([hb[:, :HW] + alo[...], hb[:, HW:] + ahi[...]], axis=1)
    h2 = jnp.dot(xa, wn[...], preferred_element_type=jnp.float32) + bn[...]
    bvec = bt[...].reshape(NB)
    oh = (bvec[:, None] == lax.broadcasted_iota(jnp.int32, (NB, G), 1)
          ).astype(jnp.float32)
    contrib = lax.dot_general(oh, h2, (((0,), (0,)), ((), ())),
                              preferred_element_type=jnp.float32)

    @pl.when(i == 0)
    def _():
        acc[...] = contrib

    @pl.when(i > 0)
    def _():
        acc[...] = acc[...] + contrib

    @pl.when(i == (N // NB) - 1)
    def _():
        pooled = acc[...]
        r1 = jnp.maximum(
            jnp.dot(pooled, wr1[...], preferred_element_type=jnp.float32)
            + br1[...], 0.0)
        out[...] = (jnp.dot(r1, wr2[...], preferred_element_type=jnp.float32)
                    + br2[...])


def _final(h, agg_lo, agg_hi, W_n, b_n, batch3, W_r1, b_r1, W_r2, b_r2):
    grid = (N // NB,)
    return pl.pallas_call(
        _final_body,
        grid=grid,
        in_specs=[
            pl.BlockSpec((NB, D), lambda i: (i, 0)),
            pl.BlockSpec((NB, HW), lambda i: (i, 0)),
            pl.BlockSpec((NB, HW), lambda i: (i, 0)),
            pl.BlockSpec((D, H), lambda i: (0, 0)),
            pl.BlockSpec((1, H), lambda i: (0, 0)),
            pl.BlockSpec((1, 1, NB), lambda i: (i, 0, 0)),
            pl.BlockSpec((H, H // 2), lambda i: (0, 0)),
            pl.BlockSpec((1, H // 2), lambda i: (0, 0)),
            pl.BlockSpec((H // 2, 1), lambda i: (0, 0)),
            pl.BlockSpec((1, 1), lambda i: (0, 0)),
        ],
        out_specs=pl.BlockSpec((G, 1), lambda i: (0, 0)),
        out_shape=jax.ShapeDtypeStruct((G, 1), jnp.float32),
        scratch_shapes=[pltpu.VMEM((G, H), jnp.float32)],
    )(h, agg_lo, agg_hi, W_n, b_n.reshape(1, H), batch3,
      W_r1, b_r1.reshape(1, H // 2), W_r2, b_r2.reshape(1, 1))


def kernel(x, edge_index, edge_attr, batch, W_e1, b_e1, W_n1, b_n1,
           W_e2, b_e2, W_n2, b_n2, W_r1, b_r1, W_r2, b_r2):
    perm = jnp.argsort(edge_index[0])
    src = edge_index[0][perm]
    dst = edge_index[1][perm]
    edge_attr = edge_attr[perm]
    pad = EPAD - E
    src_p = jnp.concatenate([src, jnp.zeros((pad,), jnp.int32)])
    dst_p = jnp.concatenate([dst, jnp.full((pad,), N, jnp.int32)])
    ea_p = jnp.concatenate([edge_attr, jnp.zeros((pad, ED), jnp.float32)])
    src3 = src_p.reshape(NSUB * NGR, IGRP, CHUNK)
    dst3 = dst_p.reshape(NSUB * NGR, IGRP, CHUNK)

    e1i, e2i = _edge_lin(ea_p, W_e1, b_e1, W_e2, b_e2)

    a1_lo, a1_hi = _sc_msg(x.reshape(2 * N, HW), e1i, src3, dst3)
    h = _node_lin(x, a1_lo, a1_hi, W_n1, b_n1)

    a2_lo, a2_hi = _sc_msg(h.reshape(2 * N, HW), e2i, src3, dst3)
    out = _final(h, a2_lo, a2_hi, W_n2, b_n2, batch.reshape(N // NB, 1, NB),
                 W_r1, b_r1, W_r2, b_r2)
    return out.reshape(G)


# speedup vs baseline: 1.3166x; 1.3166x over previous
import functools

import jax
import jax.numpy as jnp
from jax import lax
from jax.experimental import pallas as pl
from jax.experimental.pallas import tpu as pltpu
from jax.experimental.pallas import tpu_sc as plsc

N = 10000
E = 160000
D = 256
ED = 16
H = 256
G = 64

NCORE = 2
NSUB = 16
HW = 128
CHUNK = 32
NC = 320
IGRP = 16
NGR = NC // IGRP
EPAD = NSUB * NC * CHUNK
NCH = EPAD // CHUNK
NSP = 10112
ZROWS = NSP // NSUB
OROWS = 632
OROWS_LAST = N - (NSUB - 1) * OROWS
EB = 2048
NB = 1000


def _edge_lin_body(ea, we, be, o1):
    blk = ea[...]
    e1 = jnp.dot(blk, we[0], preferred_element_type=jnp.float32) + be[0]
    o1[...] = e1.reshape(EB // CHUNK, CHUNK, HW)


def _edge_lin(ea_p, W_e, b_e):
    nprog = EPAD // EB
    grid = (nprog, NCORE)
    out_shape = jax.ShapeDtypeStruct((NCORE * NCH, CHUNK, HW), jnp.float32)
    return pl.pallas_call(
        _edge_lin_body,
        grid=grid,
        in_specs=[
            pl.BlockSpec((EB, ED), lambda i, c: (i, 0)),
            pl.BlockSpec((1, ED, HW), lambda i, c: (c, 0, 0)),
            pl.BlockSpec((1, 1, HW), lambda i, c: (c, 0, 0)),
        ],
        out_specs=pl.BlockSpec((EB // CHUNK, CHUNK, HW),
                               lambda i, c: (c * nprog + i, 0, 0)),
        out_shape=out_shape,
    )(ea_p, W_e.reshape(ED, NCORE, HW).transpose(1, 0, 2),
      b_e.reshape(NCORE, 1, HW))


def _sc_msg_body(xi, e4, src3, dst3, out_lo, out_hi,
                 idx_a, idx_b, dst_v,
                 eb0, eb1, eb2, eb3, xb0, xb1, xb2, xb3, mb0, mb1,
                 agg_sh,
                 se0, se1, se2, se3, ss0, ss1):
    c = lax.axis_index("c")
    sid = lax.axis_index("s")
    ebs, xbs = (eb0, eb1, eb2, eb3), (xb0, xb1, xb2, xb3)
    ses = (se0, se1, se2, se3)
    mbs, sss = (mb0, mb1), (ss0, ss1)
    ebase = c * NCH + sid * NC

    @plsc.parallel_loop(0, CHUNK)
    def _(r):
        for k in range(HW // 16):
            mb0[r, pl.ds(k * 16, 16)] = jnp.zeros((16,), jnp.float32)
    for z in range(ZROWS // CHUNK):
        pltpu.sync_copy(mb0, agg_sh.at[pl.ds(sid * ZROWS + z * CHUNK, CHUNK)])
    zrem = ZROWS % CHUNK
    if zrem:
        pltpu.sync_copy(
            mb0.at[pl.ds(0, zrem)],
            agg_sh.at[pl.ds(sid * ZROWS + (ZROWS // CHUNK) * CHUNK, zrem)])

    def load_idx(buf, grp):
        pltpu.sync_copy(src3.at[sid * NGR + grp], buf)

        @plsc.parallel_loop(0, IGRP)
        def _(i):
            for k in range(CHUNK // 16):
                sl = pl.ds(k * 16, 16)
                buf[i, sl] = buf[i, sl] * 2 + c

    def wait_e(par):
        pltpu.make_async_copy(e4.at[ebase], ebs[par], ses[par]).wait()

    def wait_x(par):
        pltpu.make_async_copy(xi.at[pl.ds(0, CHUNK)], xbs[par], ses[par]).wait()

    def wait_s(par):
        pltpu.make_async_copy(e4.at[ebase], mbs[par], sss[par]).wait()

    def step(grp, lcl, par4, par2, wait_sc, pf_row, pf_guard):
        wait_e(par4)
        wait_x(par4)
        if wait_sc:
            wait_s(par2)

        eb, xb, mb = ebs[par4], xbs[par4], mbs[par2]

        @plsc.parallel_loop(0, CHUNK, unroll=2)
        def _(r):
            for k in range(HW // 16):
                sl = pl.ds(k * 16, 16)
                mb[r, sl] = jnp.maximum(eb[r, sl] + xb[r, sl], 0.0)

        pltpu.async_copy(mb, agg_sh.at[dst_v.at[lcl]], sss[par2], add=True)

        if pf_row is not None:
            j2 = grp * IGRP + lcl + 4

            @pl.when(pf_guard)
            def _():
                pltpu.async_copy(e4.at[ebase + j2], ebs[par4], ses[par4])
                pltpu.async_copy(xi.at[pf_row], xbs[par4], ses[par4])

    load_idx(idx_a, 0)
    pltpu.sync_copy(dst3.at[sid * NGR], dst_v)

    plsc.subcore_barrier()

    for q in range(4):
        pltpu.async_copy(e4.at[ebase + q], ebs[q], ses[q])
        pltpu.async_copy(xi.at[idx_a.at[q]], xbs[q], ses[q])

    def quad(grp, base, gcur, wait_sc_head=True, pf_bufs=None,
             pf_guard=None):
        for q in range(4):
            lcl = base + q
            wsc = wait_sc_head if (q < 2) else True
            if pf_bufs is None:
                step(grp, lcl, q, q % 2, wsc, gcur.at[base + 4 + q],
                     jnp.bool_(True))
            else:
                step(grp, lcl, q, q % 2, wsc, pf_bufs[q], pf_guard)

    def run_group(grp, gcur, gnxt, last):
        quad(grp, 0, gcur, wait_sc_head=False)

        def _quad(t, _):
            quad(grp, 4 * t, gcur)
            return 0
        lax.fori_loop(1, IGRP // 4 - 1, _quad, 0)

        quad(grp, IGRP - 4, gcur,
             pf_bufs=[gnxt.at[q] for q in range(4)],
             pf_guard=jnp.logical_not(last))
        wait_s(0)
        wait_s(1)

        @pl.when(jnp.logical_not(last))
        def _():
            pltpu.sync_copy(dst3.at[sid * NGR + grp + 1], dst_v)

    def _gg(gg, _):
        g0 = 2 * gg
        load_idx(idx_b, g0 + 1)
        run_group(g0, idx_a, idx_b, jnp.bool_(False))

        @pl.when(gg < NGR // 2 - 1)
        def _():
            load_idx(idx_a, g0 + 2)
        run_group(g0 + 1, idx_b, idx_a, gg >= NGR // 2 - 1)
        return 0
    lax.fori_loop(0, NGR // 2, _gg, 0)

    plsc.subcore_barrier()

    base = sid * OROWS
    for cc, out in ((0, out_lo), (1, out_hi)):
        @pl.when(jnp.logical_and(c == cc, sid < NSUB - 1))
        def _(out=out):
            pltpu.sync_copy(agg_sh.at[pl.ds(base, OROWS)],
                            out.at[pl.ds(base, OROWS)])

        @pl.when(jnp.logical_and(c == cc, sid == NSUB - 1))
        def _(out=out):
            pltpu.sync_copy(agg_sh.at[pl.ds(base, OROWS_LAST)],
                            out.at[pl.ds(base, OROWS_LAST)])


_sc_msg = functools.partial(
    pl.kernel,
    out_type=(jax.ShapeDtypeStruct((N, HW), jnp.float32),
              jax.ShapeDtypeStruct((N, HW), jnp.float32)),
    mesh=plsc.VectorSubcoreMesh(core_axis_name="c", subcore_axis_name="s"),
    scratch_types=(
        [pltpu.VMEM((IGRP, CHUNK), jnp.int32) for _ in range(3)]
        + [pltpu.VMEM((CHUNK, HW), jnp.float32) for _ in range(10)]
        + [pltpu.VMEM_SHARED((NSP, HW), jnp.float32)]
        + [pltpu.SemaphoreType.DMA for _ in range(6)]
    ),
)(_sc_msg_body)


def _node_lin_body(xr, alo, ahi, wn, bn, out):
    xb = xr[...]
    xa = jnp.concatenate([xb[:, :HW] + alo[...], xb[:, HW:] + ahi[...]], axis=1)
    h = jnp.dot(xa, wn[...], preferred_element_type=jnp.float32) + bn[0]
    out[...] = h


def _node_lin(x, agg_lo, agg_hi, W_n, b_n):
    grid = (N // NB, NCORE)
    return pl.pallas_call(
        _node_lin_body,
        grid=grid,
        in_specs=[
            pl.BlockSpec((NB, D), lambda i, c: (i, 0)),
            pl.BlockSpec((NB, HW), lambda i, c: (i, 0)),
            pl.BlockSpec((NB, HW), lambda i, c: (i, 0)),
            pl.BlockSpec((D, HW), lambda i, c: (0, c)),
            pl.BlockSpec((1, 1, HW), lambda i, c: (c, 0, 0)),
        ],
        out_specs=pl.BlockSpec((NB, HW), lambda i, c: (i, c)),
        out_shape=jax.ShapeDtypeStruct((N, D), jnp.float32),
    )(x, agg_lo, agg_hi, W_n, b_n.reshape(NCORE, 1, HW))


def _final_body(h, alo, ahi, wn, bn, bt, wr1, br1, wr2, br2, out, acc):
    i = pl.program_id(0)
    hb = h[...]
    xa = jnp.concatenate([hb[:, :HW] + alo[...], hb[:, HW:] + ahi[...]], axis=1)
    h2 = jnp.dot(xa, wn[...], preferred_element_type=jnp.float32) + bn[...]
    bvec = bt[...].reshape(NB)
    oh = (bvec[:, None] == lax.broadcasted_iota(jnp.int32, (NB, G), 1)
          ).astype(jnp.float32)
    contrib = lax.dot_general(oh, h2, (((0,), (0,)), ((), ())),
                              preferred_element_type=jnp.float32)

    @pl.when(i == 0)
    def _():
        acc[...] = contrib

    @pl.when(i > 0)
    def _():
        acc[...] = acc[...] + contrib

    @pl.when(i == (N // NB) - 1)
    def _():
        pooled = acc[...]
        r1 = jnp.maximum(
            jnp.dot(pooled, wr1[...], preferred_element_type=jnp.float32)
            + br1[...], 0.0)
        out[...] = (jnp.dot(r1, wr2[...], preferred_element_type=jnp.float32)
                    + br2[...])


def _final(h, agg_lo, agg_hi, W_n, b_n, batch3, W_r1, b_r1, W_r2, b_r2):
    grid = (N // NB,)
    return pl.pallas_call(
        _final_body,
        grid=grid,
        in_specs=[
            pl.BlockSpec((NB, D), lambda i: (i, 0)),
            pl.BlockSpec((NB, HW), lambda i: (i, 0)),
            pl.BlockSpec((NB, HW), lambda i: (i, 0)),
            pl.BlockSpec((D, H), lambda i: (0, 0)),
            pl.BlockSpec((1, H), lambda i: (0, 0)),
            pl.BlockSpec((1, 1, NB), lambda i: (i, 0, 0)),
            pl.BlockSpec((H, H // 2), lambda i: (0, 0)),
            pl.BlockSpec((1, H // 2), lambda i: (0, 0)),
            pl.BlockSpec((H // 2, 1), lambda i: (0, 0)),
            pl.BlockSpec((1, 1), lambda i: (0, 0)),
        ],
        out_specs=pl.BlockSpec((G, 1), lambda i: (0, 0)),
        out_shape=jax.ShapeDtypeStruct((G, 1), jnp.float32),
        scratch_shapes=[pltpu.VMEM((G, H), jnp.float32)],
    )(h, agg_lo, agg_hi, W_n, b_n.reshape(1, H), batch3,
      W_r1, b_r1.reshape(1, H // 2), W_r2, b_r2.reshape(1, 1))


def kernel(x, edge_index, edge_attr, batch, W_e1, b_e1, W_n1, b_n1,
           W_e2, b_e2, W_n2, b_n2, W_r1, b_r1, W_r2, b_r2):
    src = edge_index[0]
    dst = edge_index[1]
    pad = EPAD - E
    src_p = jnp.concatenate([src, jnp.zeros((pad,), jnp.int32)])
    dst_p = jnp.concatenate([dst, jnp.full((pad,), N, jnp.int32)])
    ea_p = jnp.concatenate([edge_attr, jnp.zeros((pad, ED), jnp.float32)])
    src3 = src_p.reshape(NSUB * NGR, IGRP, CHUNK)
    dst3 = dst_p.reshape(NSUB * NGR, IGRP, CHUNK)

    e1i = _edge_lin(ea_p, W_e1, b_e1)

    a1_lo, a1_hi = _sc_msg(x.reshape(2 * N, HW), e1i, src3, dst3)
    e2i = _edge_lin(ea_p, W_e2, b_e2)
    h = _node_lin(x, a1_lo, a1_hi, W_n1, b_n1)

    a2_lo, a2_hi = _sc_msg(h.reshape(2 * N, HW), e2i, src3, dst3)
    out = _final(h, a2_lo, a2_hi, W_n2, b_n2, batch.reshape(N // NB, 1, NB),
                 W_r1, b_r1, W_r2, b_r2)
    return out.reshape(G)


# speedup vs baseline: 1.3367x; 1.0153x over previous
import functools

import jax
import jax.numpy as jnp
from jax import lax
from jax.experimental import pallas as pl
from jax.experimental.pallas import tpu as pltpu
from jax.experimental.pallas import tpu_sc as plsc

N = 10000
E = 160000
D = 256
ED = 16
H = 256
G = 64

NCORE = 2
NSUB = 16
HW = 128
CHUNK = 32
NC = 320
IGRP = 16
NGR = NC // IGRP
EPAD = NSUB * NC * CHUNK
NCH = EPAD // CHUNK
NSP = 10112
ZROWS = NSP // NSUB
OROWS = 632
OROWS_LAST = N - (NSUB - 1) * OROWS
EB = 2048
NB = 1000


def _edge_lin_body(ea, we1, be1, we2, be2, o1, o2):
    blk = ea[...]
    e1 = jnp.dot(blk, we1[0], preferred_element_type=jnp.float32) + be1[0]
    e2 = jnp.dot(blk, we2[0], preferred_element_type=jnp.float32) + be2[0]
    o1[...] = e1.reshape(EB // CHUNK, CHUNK, HW)
    o2[...] = e2.reshape(EB // CHUNK, CHUNK, HW)


def _edge_lin(ea_p, W_e1, b_e1, W_e2, b_e2):
    nprog = EPAD // EB
    grid = (nprog, NCORE)
    out_shape = jax.ShapeDtypeStruct((NCORE * NCH, CHUNK, HW), jnp.float32)
    return pl.pallas_call(
        _edge_lin_body,
        grid=grid,
        in_specs=[
            pl.BlockSpec((EB, ED), lambda i, c: (i, 0)),
            pl.BlockSpec((1, ED, HW), lambda i, c: (c, 0, 0)),
            pl.BlockSpec((1, 1, HW), lambda i, c: (c, 0, 0)),
            pl.BlockSpec((1, ED, HW), lambda i, c: (c, 0, 0)),
            pl.BlockSpec((1, 1, HW), lambda i, c: (c, 0, 0)),
        ],
        out_specs=[
            pl.BlockSpec((EB // CHUNK, CHUNK, HW),
                         lambda i, c: (c * nprog + i, 0, 0)),
            pl.BlockSpec((EB // CHUNK, CHUNK, HW),
                         lambda i, c: (c * nprog + i, 0, 0)),
        ],
        out_shape=[out_shape, out_shape],
    )(ea_p, W_e1.reshape(ED, NCORE, HW).transpose(1, 0, 2),
      b_e1.reshape(NCORE, 1, HW),
      W_e2.reshape(ED, NCORE, HW).transpose(1, 0, 2),
      b_e2.reshape(NCORE, 1, HW))


def _sc_msg_body(xi, e4, src3, dst3, out_lo, out_hi,
                 idx_a, idx_b, dst_v,
                 eb0, eb1, eb2, eb3, xb0, xb1, xb2, xb3, mb0, mb1,
                 agg_sh,
                 se0, se1, se2, se3, ss0, ss1):
    c = lax.axis_index("c")
    sid = lax.axis_index("s")
    ebs, xbs = (eb0, eb1, eb2, eb3), (xb0, xb1, xb2, xb3)
    ses = (se0, se1, se2, se3)
    mbs, sss = (mb0, mb1), (ss0, ss1)
    ebase = c * NCH + sid * NC

    @plsc.parallel_loop(0, CHUNK)
    def _(r):
        for k in range(HW // 16):
            mb0[r, pl.ds(k * 16, 16)] = jnp.zeros((16,), jnp.float32)
    for z in range(ZROWS // CHUNK):
        pltpu.sync_copy(mb0, agg_sh.at[pl.ds(sid * ZROWS + z * CHUNK, CHUNK)])
    zrem = ZROWS % CHUNK
    if zrem:
        pltpu.sync_copy(
            mb0.at[pl.ds(0, zrem)],
            agg_sh.at[pl.ds(sid * ZROWS + (ZROWS // CHUNK) * CHUNK, zrem)])

    def load_idx(buf, grp):
        pltpu.sync_copy(src3.at[sid * NGR + grp], buf)

        @plsc.parallel_loop(0, IGRP)
        def _(i):
            for k in range(CHUNK // 16):
                sl = pl.ds(k * 16, 16)
                buf[i, sl] = buf[i, sl] * 2 + c

    def wait_e(par):
        pltpu.make_async_copy(e4.at[ebase], ebs[par], ses[par]).wait()

    def wait_x(par):
        pltpu.make_async_copy(xi.at[pl.ds(0, CHUNK)], xbs[par], ses[par]).wait()

    def wait_s(par):
        pltpu.make_async_copy(e4.at[ebase], mbs[par], sss[par]).wait()

    def step(grp, lcl, par4, par2, wait_sc, pf_row, pf_guard):
        wait_e(par4)
        wait_x(par4)
        if wait_sc:
            wait_s(par2)

        eb, xb, mb = ebs[par4], xbs[par4], mbs[par2]

        @plsc.parallel_loop(0, CHUNK, unroll=4)
        def _(r):
            for k in range(HW // 16):
                sl = pl.ds(k * 16, 16)
                mb[r, sl] = jnp.maximum(eb[r, sl] + xb[r, sl], 0.0)

        pltpu.async_copy(mb, agg_sh.at[dst_v.at[lcl]], sss[par2], add=True)

        if pf_row is not None:
            j2 = grp * IGRP + lcl + 4

            @pl.when(pf_guard)
            def _():
                pltpu.async_copy(e4.at[ebase + j2], ebs[par4], ses[par4])
                pltpu.async_copy(xi.at[pf_row], xbs[par4], ses[par4])

    load_idx(idx_a, 0)
    pltpu.sync_copy(dst3.at[sid * NGR], dst_v)

    plsc.subcore_barrier()

    for q in range(4):
        pltpu.async_copy(e4.at[ebase + q], ebs[q], ses[q])
        pltpu.async_copy(xi.at[idx_a.at[q]], xbs[q], ses[q])

    def quad(grp, base, gcur, wait_sc_head=True, pf_bufs=None,
             pf_guard=None):
        for q in range(4):
            lcl = base + q
            wsc = wait_sc_head if (q < 2) else True
            if pf_bufs is None:
                step(grp, lcl, q, q % 2, wsc, gcur.at[base + 4 + q],
                     jnp.bool_(True))
            else:
                step(grp, lcl, q, q % 2, wsc, pf_bufs[q], pf_guard)

    def run_group(grp, gcur, gnxt, last):
        quad(grp, 0, gcur, wait_sc_head=False)

        def _quad(t, _):
            quad(grp, 4 * t, gcur)
            return 0
        lax.fori_loop(1, IGRP // 4 - 1, _quad, 0)

        quad(grp, IGRP - 4, gcur,
             pf_bufs=[gnxt.at[q] for q in range(4)],
             pf_guard=jnp.logical_not(last))
        wait_s(0)
        wait_s(1)

        @pl.when(jnp.logical_not(last))
        def _():
            pltpu.sync_copy(dst3.at[sid * NGR + grp + 1], dst_v)

    def _gg(gg, _):
        g0 = 2 * gg
        load_idx(idx_b, g0 + 1)
        run_group(g0, idx_a, idx_b, jnp.bool_(False))

        @pl.when(gg < NGR // 2 - 1)
        def _():
            load_idx(idx_a, g0 + 2)
        run_group(g0 + 1, idx_b, idx_a, gg >= NGR // 2 - 1)
        return 0
    lax.fori_loop(0, NGR // 2, _gg, 0)

    plsc.subcore_barrier()

    base = sid * OROWS
    for cc, out in ((0, out_lo), (1, out_hi)):
        @pl.when(jnp.logical_and(c == cc, sid < NSUB - 1))
        def _(out=out):
            pltpu.sync_copy(agg_sh.at[pl.ds(base, OROWS)],
                            out.at[pl.ds(base, OROWS)])

        @pl.when(jnp.logical_and(c == cc, sid == NSUB - 1))
        def _(out=out):
            pltpu.sync_copy(agg_sh.at[pl.ds(base, OROWS_LAST)],
                            out.at[pl.ds(base, OROWS_LAST)])


_sc_msg = functools.partial(
    pl.kernel,
    out_type=(jax.ShapeDtypeStruct((N, HW), jnp.float32),
              jax.ShapeDtypeStruct((N, HW), jnp.float32)),
    mesh=plsc.VectorSubcoreMesh(core_axis_name="c", subcore_axis_name="s"),
    scratch_types=(
        [pltpu.VMEM((IGRP, CHUNK), jnp.int32) for _ in range(3)]
        + [pltpu.VMEM((CHUNK, HW), jnp.float32) for _ in range(10)]
        + [pltpu.VMEM_SHARED((NSP, HW), jnp.float32)]
        + [pltpu.SemaphoreType.DMA for _ in range(6)]
    ),
)(_sc_msg_body)


def _node_lin_body(xr, alo, ahi, wn, bn, out):
    xb = xr[...]
    xa = jnp.concatenate([xb[:, :HW] + alo[...], xb[:, HW:] + ahi[...]], axis=1)
    h = jnp.dot(xa, wn[...], preferred_element_type=jnp.float32) + bn[0]
    out[...] = h


def _node_lin(x, agg_lo, agg_hi, W_n, b_n):
    grid = (N // NB, NCORE)
    return pl.pallas_call(
        _node_lin_body,
        grid=grid,
        in_specs=[
            pl.BlockSpec((NB, D), lambda i, c: (i, 0)),
            pl.BlockSpec((NB, HW), lambda i, c: (i, 0)),
            pl.BlockSpec((NB, HW), lambda i, c: (i, 0)),
            pl.BlockSpec((D, HW), lambda i, c: (0, c)),
            pl.BlockSpec((1, 1, HW), lambda i, c: (c, 0, 0)),
        ],
        out_specs=pl.BlockSpec((NB, HW), lambda i, c: (i, c)),
        out_shape=jax.ShapeDtypeStruct((N, D), jnp.float32),
    )(x, agg_lo, agg_hi, W_n, b_n.reshape(NCORE, 1, HW))


def _final_body(h, alo, ahi, wn, bn, bt, wr1, br1, wr2, br2, out, acc):
    i = pl.program_id(0)
    hb = h[...]
    xa = jnp.concatenate([hb[:, :HW] + alo[...], hb[:, HW:] + ahi[...]], axis=1)
    h2 = jnp.dot(xa, wn[...], preferred_element_type=jnp.float32) + bn[...]
    bvec = bt[...].reshape(NB)
    oh = (bvec[:, None] == lax.broadcasted_iota(jnp.int32, (NB, G), 1)
          ).astype(jnp.float32)
    contrib = lax.dot_general(oh, h2, (((0,), (0,)), ((), ())),
                              preferred_element_type=jnp.float32)

    @pl.when(i == 0)
    def _():
        acc[...] = contrib

    @pl.when(i > 0)
    def _():
        acc[...] = acc[...] + contrib

    @pl.when(i == (N // NB) - 1)
    def _():
        pooled = acc[...]
        r1 = jnp.maximum(
            jnp.dot(pooled, wr1[...], preferred_element_type=jnp.float32)
            + br1[...], 0.0)
        out[...] = (jnp.dot(r1, wr2[...], preferred_element_type=jnp.float32)
                    + br2[...])


def _final(h, agg_lo, agg_hi, W_n, b_n, batch3, W_r1, b_r1, W_r2, b_r2):
    grid = (N // NB,)
    return pl.pallas_call(
        _final_body,
        grid=grid,
        in_specs=[
            pl.BlockSpec((NB, D), lambda i: (i, 0)),
            pl.BlockSpec((NB, HW), lambda i: (i, 0)),
            pl.BlockSpec((NB, HW), lambda i: (i, 0)),
            pl.BlockSpec((D, H), lambda i: (0, 0)),
            pl.BlockSpec((1, H), lambda i: (0, 0)),
            pl.BlockSpec((1, 1, NB), lambda i: (i, 0, 0)),
            pl.BlockSpec((H, H // 2), lambda i: (0, 0)),
            pl.BlockSpec((1, H // 2), lambda i: (0, 0)),
            pl.BlockSpec((H // 2, 1), lambda i: (0, 0)),
            pl.BlockSpec((1, 1), lambda i: (0, 0)),
        ],
        out_specs=pl.BlockSpec((G, 1), lambda i: (0, 0)),
        out_shape=jax.ShapeDtypeStruct((G, 1), jnp.float32),
        scratch_shapes=[pltpu.VMEM((G, H), jnp.float32)],
    )(h, agg_lo, agg_hi, W_n, b_n.reshape(1, H), batch3,
      W_r1, b_r1.reshape(1, H // 2), W_r2, b_r2.reshape(1, 1))


def kernel(x, edge_index, edge_attr, batch, W_e1, b_e1, W_n1, b_n1,
           W_e2, b_e2, W_n2, b_n2, W_r1, b_r1, W_r2, b_r2):
    src = edge_index[0]
    dst = edge_index[1]
    pad = EPAD - E
    src_p = jnp.concatenate([src, jnp.zeros((pad,), jnp.int32)])
    dst_p = jnp.concatenate([dst, jnp.full((pad,), N, jnp.int32)])
    ea_p = jnp.concatenate([edge_attr, jnp.zeros((pad, ED), jnp.float32)])
    src3 = src_p.reshape(NSUB * NGR, IGRP, CHUNK)
    dst3 = dst_p.reshape(NSUB * NGR, IGRP, CHUNK)

    e1i, e2i = _edge_lin(ea_p, W_e1, b_e1, W_e2, b_e2)

    a1_lo, a1_hi = _sc_msg(x.reshape(2 * N, HW), e1i, src3, dst3)
    h = _node_lin(x, a1_lo, a1_hi, W_n1, b_n1)

    a2_lo, a2_hi = _sc_msg(h.reshape(2 * N, HW), e2i, src3, dst3)
    out = _final(h, a2_lo, a2_hi, W_n2, b_n2, batch.reshape(N // NB, 1, NB),
                 W_r1, b_r1, W_r2, b_r2)
    return out.reshape(G)


# speedup vs baseline: 1.3461x; 1.0070x over previous
import functools

import jax
import jax.numpy as jnp
from jax import lax
from jax.experimental import pallas as pl
from jax.experimental.pallas import tpu as pltpu
from jax.experimental.pallas import tpu_sc as plsc

N = 10000
E = 160000
D = 256
ED = 16
H = 256
G = 64

NCORE = 2
NSUB = 16
HW = 128
CHUNK = 32
NC = 320
IGRP = 16
NGR = NC // IGRP
EPAD = NSUB * NC * CHUNK
NCH = EPAD // CHUNK
NSP = 10112
ZROWS = NSP // NSUB
OROWS = 632
OROWS_LAST = N - (NSUB - 1) * OROWS
EB = 2048
NB = 1000


def _edge_lin_body(ea, we1, be1, we2, be2, o1, o2):
    blk = ea[...]
    e1 = jnp.dot(blk, we1[0], preferred_element_type=jnp.float32) + be1[0]
    e2 = jnp.dot(blk, we2[0], preferred_element_type=jnp.float32) + be2[0]
    o1[...] = e1.reshape(EB // CHUNK, CHUNK, HW)
    o2[...] = e2.reshape(EB // CHUNK, CHUNK, HW)


def _edge_lin(ea_p, W_e1, b_e1, W_e2, b_e2):
    nprog = EPAD // EB
    grid = (nprog, NCORE)
    out_shape = jax.ShapeDtypeStruct((NCORE * NCH, CHUNK, HW), jnp.float32)
    return pl.pallas_call(
        _edge_lin_body,
        grid=grid,
        in_specs=[
            pl.BlockSpec((EB, ED), lambda i, c: (i, 0)),
            pl.BlockSpec((1, ED, HW), lambda i, c: (c, 0, 0)),
            pl.BlockSpec((1, 1, HW), lambda i, c: (c, 0, 0)),
            pl.BlockSpec((1, ED, HW), lambda i, c: (c, 0, 0)),
            pl.BlockSpec((1, 1, HW), lambda i, c: (c, 0, 0)),
        ],
        out_specs=[
            pl.BlockSpec((EB // CHUNK, CHUNK, HW),
                         lambda i, c: (c * nprog + i, 0, 0)),
            pl.BlockSpec((EB // CHUNK, CHUNK, HW),
                         lambda i, c: (c * nprog + i, 0, 0)),
        ],
        out_shape=[out_shape, out_shape],
    )(ea_p, W_e1.reshape(ED, NCORE, HW).transpose(1, 0, 2),
      b_e1.reshape(NCORE, 1, HW),
      W_e2.reshape(ED, NCORE, HW).transpose(1, 0, 2),
      b_e2.reshape(NCORE, 1, HW))


def _sc_msg_body(xi, e4, src3, dst3, out_lo, out_hi,
                 idx_a, idx_b, dst_v,
                 eb0, eb1, eb2, eb3, xb0, xb1, xb2, xb3, mb0, mb1,
                 agg_sh,
                 se0, se1, se2, se3, ss0, ss1):
    c = lax.axis_index("c")
    sid = lax.axis_index("s")
    ebs, xbs = (eb0, eb1, eb2, eb3), (xb0, xb1, xb2, xb3)
    ses = (se0, se1, se2, se3)
    mbs, sss = (mb0, mb1), (ss0, ss1)
    ebase = c * NCH + sid * NC

    @plsc.parallel_loop(0, CHUNK)
    def _(r):
        for k in range(HW // 16):
            mb0[r, pl.ds(k * 16, 16)] = jnp.zeros((16,), jnp.float32)
    for z in range(ZROWS // CHUNK):
        pltpu.sync_copy(mb0, agg_sh.at[pl.ds(sid * ZROWS + z * CHUNK, CHUNK)])
    zrem = ZROWS % CHUNK
    if zrem:
        pltpu.sync_copy(
            mb0.at[pl.ds(0, zrem)],
            agg_sh.at[pl.ds(sid * ZROWS + (ZROWS // CHUNK) * CHUNK, zrem)])

    def load_idx(buf, grp):
        pltpu.sync_copy(src3.at[sid * NGR + grp], buf)

        @plsc.parallel_loop(0, IGRP)
        def _(i):
            for k in range(CHUNK // 16):
                sl = pl.ds(k * 16, 16)
                buf[i, sl] = buf[i, sl] * 2 + c

    def wait_e(par):
        pltpu.make_async_copy(e4.at[ebase], ebs[par], ses[par]).wait()

    def wait_x(par):
        pltpu.make_async_copy(xi.at[pl.ds(0, CHUNK)], xbs[par], ses[par]).wait()

    def wait_s(par):
        pltpu.make_async_copy(e4.at[ebase], mbs[par], sss[par]).wait()

    def step(grp, lcl, par4, par2, wait_sc, pf_row, pf_guard):
        wait_e(par4)
        wait_x(par4)
        if wait_sc:
            wait_s(par2)

        eb, xb, mb = ebs[par4], xbs[par4], mbs[par2]

        @plsc.parallel_loop(0, CHUNK, unroll=2)
        def _(r):
            for k in range(HW // 16):
                sl = pl.ds(k * 16, 16)
                mb[r, sl] = jnp.maximum(eb[r, sl] + xb[r, sl], 0.0)

        pltpu.async_copy(mb, agg_sh.at[dst_v.at[lcl]], sss[par2], add=True)

        if pf_row is not None:
            j2 = grp * IGRP + lcl + 4

            @pl.when(pf_guard)
            def _():
                pltpu.async_copy(e4.at[ebase + j2], ebs[par4], ses[par4])
                pltpu.async_copy(xi.at[pf_row], xbs[par4], ses[par4])

    load_idx(idx_a, 0)
    pltpu.sync_copy(dst3.at[sid * NGR], dst_v)

    plsc.subcore_barrier()

    for q in range(4):
        pltpu.async_copy(e4.at[ebase + q], ebs[q], ses[q])
        pltpu.async_copy(xi.at[idx_a.at[q]], xbs[q], ses[q])

    def quad(grp, base, gcur, wait_sc_head=True, pf_bufs=None,
             pf_guard=None):
        for q in range(4):
            lcl = base + q
            wsc = wait_sc_head if (q < 2) else True
            if pf_bufs is None:
                step(grp, lcl, q, q % 2, wsc, gcur.at[base + 4 + q],
                     jnp.bool_(True))
            else:
                step(grp, lcl, q, q % 2, wsc, pf_bufs[q], pf_guard)

    def run_group(grp, gcur, gnxt, last):
        quad(grp, 0, gcur, wait_sc_head=False)

        def _quad(t, _):
            quad(grp, 4 * t, gcur)
            return 0
        lax.fori_loop(1, IGRP // 4 - 1, _quad, 0)

        quad(grp, IGRP - 4, gcur,
             pf_bufs=[gnxt.at[q] for q in range(4)],
             pf_guard=jnp.logical_not(last))
        wait_s(0)
        wait_s(1)

        @pl.when(jnp.logical_not(last))
        def _():
            pltpu.sync_copy(dst3.at[sid * NGR + grp + 1], dst_v)

    def _gg(gg, _):
        g0 = 2 * gg
        load_idx(idx_b, g0 + 1)
        run_group(g0, idx_a, idx_b, jnp.bool_(False))

        @pl.when(gg < NGR // 2 - 1)
        def _():
            load_idx(idx_a, g0 + 2)
        run_group(g0 + 1, idx_b, idx_a, gg >= NGR // 2 - 1)
        return 0
    lax.fori_loop(0, NGR // 2, _gg, 0)

    plsc.subcore_barrier()

    base = sid * OROWS
    for cc, out in ((0, out_lo), (1, out_hi)):
        @pl.when(jnp.logical_and(c == cc, sid < NSUB - 1))
        def _(out=out):
            pltpu.sync_copy(agg_sh.at[pl.ds(base, OROWS)],
                            out.at[pl.ds(base, OROWS)])

        @pl.when(jnp.logical_and(c == cc, sid == NSUB - 1))
        def _(out=out):
            pltpu.sync_copy(agg_sh.at[pl.ds(base, OROWS_LAST)],
                            out.at[pl.ds(base, OROWS_LAST)])


_sc_msg = functools.partial(
    pl.kernel,
    out_type=(jax.ShapeDtypeStruct((N, HW), jnp.float32),
              jax.ShapeDtypeStruct((N, HW), jnp.float32)),
    mesh=plsc.VectorSubcoreMesh(core_axis_name="c", subcore_axis_name="s"),
    scratch_types=(
        [pltpu.VMEM((IGRP, CHUNK), jnp.int32) for _ in range(3)]
        + [pltpu.VMEM((CHUNK, HW), jnp.float32) for _ in range(10)]
        + [pltpu.VMEM_SHARED((NSP, HW), jnp.float32)]
        + [pltpu.SemaphoreType.DMA for _ in range(6)]
    ),
)(_sc_msg_body)


def _node_lin_body(xr, alo, ahi, wn, bn, out):
    xb = xr[...]
    xa = jnp.concatenate([xb[:, :HW] + alo[...], xb[:, HW:] + ahi[...]], axis=1)
    h = jnp.dot(xa, wn[...], preferred_element_type=jnp.float32) + bn[0]
    out[...] = h


def _node_lin(x, agg_lo, agg_hi, W_n, b_n):
    grid = (N // NB, NCORE)
    return pl.pallas_call(
        _node_lin_body,
        grid=grid,
        in_specs=[
            pl.BlockSpec((NB, D), lambda i, c: (i, 0)),
            pl.BlockSpec((NB, HW), lambda i, c: (i, 0)),
            pl.BlockSpec((NB, HW), lambda i, c: (i, 0)),
            pl.BlockSpec((D, HW), lambda i, c: (0, c)),
            pl.BlockSpec((1, 1, HW), lambda i, c: (c, 0, 0)),
        ],
        out_specs=pl.BlockSpec((NB, HW), lambda i, c: (i, c)),
        out_shape=jax.ShapeDtypeStruct((N, D), jnp.float32),
    )(x, agg_lo, agg_hi, W_n, b_n.reshape(NCORE, 1, HW))


def _final_body(h, alo, ahi, wn, bn, bt, wr1, br1, wr2, br2, out, acc):
    i = pl.program_id(0)
    hb = h[...]
    xa = jnp.concatenate([hb[:, :HW] + alo[...], hb[:, HW:] + ahi[...]], axis=1)
    h2 = jnp.dot(xa, wn[...], preferred_element_type=jnp.float32) + bn[...]
    bvec = bt[...].reshape(NB)
    oh = (bvec[:, None] == lax.broadcasted_iota(jnp.int32, (NB, G), 1)
          ).astype(jnp.float32)
    contrib = lax.dot_general(oh, h2, (((0,), (0,)), ((), ())),
                              preferred_element_type=jnp.float32)

    @pl.when(i == 0)
    def _():
        acc[...] = contrib

    @pl.when(i > 0)
    def _():
        acc[...] = acc[...] + contrib

    @pl.when(i == (N // NB) - 1)
    def _():
        pooled = acc[...]
        r1 = jnp.maximum(
            jnp.dot(pooled, wr1[...], preferred_element_type=jnp.float32)
            + br1[...], 0.0)
        out[...] = (jnp.dot(r1, wr2[...], preferred_element_type=jnp.float32)
                    + br2[...])


def _final(h, agg_lo, agg_hi, W_n, b_n, batch3, W_r1, b_r1, W_r2, b_r2):
    grid = (N // NB,)
    return pl.pallas_call(
        _final_body,
        grid=grid,
        in_specs=[
            pl.BlockSpec((NB, D), lambda i: (i, 0)),
            pl.BlockSpec((NB, HW), lambda i: (i, 0)),
            pl.BlockSpec((NB, HW), lambda i: (i, 0)),
            pl.BlockSpec((D, H), lambda i: (0, 0)),
            pl.BlockSpec((1, H), lambda i: (0, 0)),
            pl.BlockSpec((1, 1, NB), lambda i: (i, 0, 0)),
            pl.BlockSpec((H, H // 2), lambda i: (0, 0)),
            pl.BlockSpec((1, H // 2), lambda i: (0, 0)),
            pl.BlockSpec((H // 2, 1), lambda i: (0, 0)),
            pl.BlockSpec((1, 1), lambda i: (0, 0)),
        ],
        out_specs=pl.BlockSpec((G, 1), lambda i: (0, 0)),
        out_shape=jax.ShapeDtypeStruct((G, 1), jnp.float32),
        scratch_shapes=[pltpu.VMEM((G, H), jnp.float32)],
    )(h, agg_lo, agg_hi, W_n, b_n.reshape(1, H), batch3,
      W_r1, b_r1.reshape(1, H // 2), W_r2, b_r2.reshape(1, 1))


def kernel(x, edge_index, edge_attr, batch, W_e1, b_e1, W_n1, b_n1,
           W_e2, b_e2, W_n2, b_n2, W_r1, b_r1, W_r2, b_r2):
    src = edge_index[0]
    dst = edge_index[1]
    pad = EPAD - E
    src_p = jnp.concatenate([src, jnp.zeros((pad,), jnp.int32)])
    dst_p = jnp.concatenate([dst, jnp.full((pad,), N, jnp.int32)])
    ea_p = jnp.concatenate([edge_attr, jnp.zeros((pad, ED), jnp.float32)])
    src3 = src_p.reshape(NSUB * NGR, IGRP, CHUNK)
    dst3 = dst_p.reshape(NSUB * NGR, IGRP, CHUNK)

    e1i, e2i = _edge_lin(ea_p, W_e1, b_e1, W_e2, b_e2)

    a1_lo, a1_hi = _sc_msg(x.reshape(2 * N, HW), e1i, src3, dst3)
    h = _node_lin(x, a1_lo, a1_hi, W_n1, b_n1)

    a2_lo, a2_hi = _sc_msg(h.reshape(2 * N, HW), e2i, src3, dst3)
    out = _final(h, a2_lo, a2_hi, W_n2, b_n2, batch.reshape(N // NB, 1, NB),
                 W_r1, b_r1, W_r2, b_r2)
    return out.reshape(G)
